# Initial kernel scaffold; baseline (speedup 1.0000x reference)
import jax
import jax.numpy as jnp
from jax.experimental import pallas as pl


def kernel(node_features, edge_features, edge_index, W_node, b_node, W_edge, b_edge, W_pred, b_pred):
    raise NotImplementedError("write your pallas kernel here")


# speedup vs baseline: 6.2297x; 6.2297x over previous
import functools

import jax
import jax.numpy as jnp
from jax import lax
from jax.experimental import pallas as pl
from jax.experimental.pallas import tpu as pltpu
from jax.experimental.pallas import tpu_sc as plsc

N = 10000
E = 320000
D = 128
H = 128

_EDGE_BLOCK = 2000

_NUM_WORKERS = 32
_EPW = E // _NUM_WORKERS
_LANES = 16


def _node_proj_body(x_ref, w_ref, b_ref, w2_ref, o_ref):
    n = jnp.dot(x_ref[...], w_ref[...], preferred_element_type=jnp.float32)
    n = jnp.maximum(n + b_ref[...], 0.0)
    o_ref[...] = jnp.dot(n, w2_ref[...], preferred_element_type=jnp.float32)


def _edge_proj_body(x_ref, w_ref, b_ref, w3_ref, bp_ref, o_ref):
    e = jnp.dot(x_ref[...], w_ref[...], preferred_element_type=jnp.float32)
    e = jnp.maximum(e + b_ref[...], 0.0)
    o_ref[...] = (
        jnp.dot(e, w3_ref[...], preferred_element_type=jnp.float32) + bp_ref[...]
    )


def _combine_body(p12_hbm, src_hbm, dst_hbm, pe_hbm, out_hbm,
                  tab_v, src_v, dst_v, pe_v, out_v):
    wid = lax.axis_index("s") * 2 + lax.axis_index("c")
    base = wid * _EPW
    pltpu.sync_copy(p12_hbm, tab_v)
    pltpu.sync_copy(src_hbm.at[pl.ds(base, _EPW)], src_v)
    pltpu.sync_copy(dst_hbm.at[pl.ds(base, _EPW)], dst_v)
    pltpu.sync_copy(pe_hbm.at[pl.ds(base, _EPW)], pe_v)

    def body(i, carry):
        o = i * _LANES
        s = src_v[pl.ds(o, _LANES)] * 2
        d = dst_v[pl.ds(o, _LANES)] * 2 + 1
        a = plsc.load_gather(tab_v, [s])
        b = plsc.load_gather(tab_v, [d])
        out_v[pl.ds(o, _LANES)] = a + b + pe_v[pl.ds(o, _LANES)]
        return carry

    lax.fori_loop(0, _EPW // _LANES, body, 0)
    pltpu.sync_copy(out_v, out_hbm.at[pl.ds(base, _EPW)])


def kernel(node_features, edge_features, edge_index, W_node, b_node,
           W_edge, b_edge, W_pred, b_pred):
    w12 = jnp.concatenate([W_pred[0:H, :], W_pred[H:2 * H, :]], axis=1)
    w3 = W_pred[2 * H:3 * H, :]

    p12 = pl.pallas_call(
        _node_proj_body,
        out_shape=jax.ShapeDtypeStruct((N, 2), jnp.float32),
    )(node_features, W_node, b_node.reshape(1, H), w12)

    pe = pl.pallas_call(
        _edge_proj_body,
        grid=(E // _EDGE_BLOCK,),
        in_specs=[
            pl.BlockSpec((_EDGE_BLOCK, D), lambda i: (i, 0)),
            pl.BlockSpec((D, H), lambda i: (0, 0)),
            pl.BlockSpec((1, H), lambda i: (0, 0)),
            pl.BlockSpec((H, 1), lambda i: (0, 0)),
            pl.BlockSpec((1, 1), lambda i: (0, 0)),
        ],
        out_specs=pl.BlockSpec((_EDGE_BLOCK, 1), lambda i: (i, 0)),
        out_shape=jax.ShapeDtypeStruct((E, 1), jnp.float32),
    )(edge_features, W_edge, b_edge.reshape(1, H), w3, b_pred.reshape(1, 1))

    combine = functools.partial(
        pl.kernel,
        out_type=jax.ShapeDtypeStruct((E,), jnp.float32),
        mesh=plsc.VectorSubcoreMesh(core_axis_name="c", subcore_axis_name="s"),
        compiler_params=pltpu.CompilerParams(needs_layout_passes=False),
        scratch_types=[
            pltpu.VMEM((2 * N,), jnp.float32),
            pltpu.VMEM((_EPW,), jnp.int32),
            pltpu.VMEM((_EPW,), jnp.int32),
            pltpu.VMEM((_EPW,), jnp.float32),
            pltpu.VMEM((_EPW,), jnp.float32),
        ],
    )(_combine_body)

    out = combine(p12.reshape(2 * N), edge_index[0], edge_index[1],
                  pe.reshape(E))
    return out.reshape(E, 1)
